# Initial kernel scaffold; baseline (speedup 1.0000x reference)
import jax
import jax.numpy as jnp
from jax.experimental import pallas as pl


def kernel(labels, logits_0, logits_1, logits_2, logits_3, logits_4, logits_5, logits_6, logits_7):
    raise NotImplementedError("write your pallas kernel here")


# speedup vs baseline: 3.1806x; 3.1806x over previous
import functools

import jax
import jax.numpy as jnp
from jax.experimental import pallas as pl
from jax.experimental.pallas import tpu as pltpu

B = 4096
L = 1000
E = 8
BB = 256


def _oracle_block(labels_ref, *refs):
    logits_refs = refs[:E]
    out_ref = refs[E]
    labels = labels_ref[...]
    iota = jax.lax.broadcasted_iota(jnp.int32, (BB, L), 1)
    lab_eq = iota == labels

    first_correct = jnp.full((BB, 1), E, dtype=jnp.int32)
    fallback = jnp.full((BB, 1), 0, dtype=jnp.int32)
    best_ll = jnp.full((BB, 1), -jnp.inf, dtype=jnp.float32)
    for e in range(E):
        x = logits_refs[e][...]
        m = jnp.max(x, axis=1, keepdims=True)
        amax = jnp.min(jnp.where(x == m, iota, L), axis=1, keepdims=True)
        correct = amax == labels
        first_correct = jnp.where(
            (first_correct == E) & correct, e, first_correct
        )
        ll = jnp.max(jnp.where(lab_eq, x, -jnp.inf), axis=1, keepdims=True)
        take = ll > best_ll
        fallback = jnp.where(take, e, fallback)
        best_ll = jnp.where(take, ll, best_ll)

    best = jnp.where(first_correct < E, first_correct, fallback)

    out = logits_refs[0][...]
    for e in range(1, E):
        out = jnp.where(best == e, logits_refs[e][...], out)
    out_ref[...] = out


@jax.jit
def kernel(labels, logits_0, logits_1, logits_2, logits_3, logits_4,
           logits_5, logits_6, logits_7):
    labels2 = labels.reshape(B, 1).astype(jnp.int32)
    logit_spec = pl.BlockSpec((BB, L), lambda i: (i, 0))
    grid = (B // BB,)
    out = pl.pallas_call(
        _oracle_block,
        grid=grid,
        in_specs=[pl.BlockSpec((BB, 1), lambda i: (i, 0))] + [logit_spec] * E,
        out_specs=logit_spec,
        out_shape=jax.ShapeDtypeStruct((B, L), jnp.float32),
        compiler_params=pltpu.CompilerParams(
            dimension_semantics=("arbitrary",),
        ),
    )(labels2, logits_0, logits_1, logits_2, logits_3, logits_4,
      logits_5, logits_6, logits_7)
    return out


# speedup vs baseline: 3.2851x; 1.0328x over previous
import functools

import jax
import jax.numpy as jnp
from jax.experimental import pallas as pl
from jax.experimental.pallas import tpu as pltpu

B = 4096
L = 1000
E = 8
BB = 256


def _oracle_block(labels_ref, *refs):
    logits_refs = refs[:E]
    out_ref = refs[E]
    labels = labels_ref[...]
    iota = jax.lax.broadcasted_iota(jnp.int32, (BB, L), 1)
    lab_eq = iota == labels
    pre = iota < labels
    ninf = jnp.float32(-jnp.inf)

    first_correct = jnp.full((BB, 1), E, dtype=jnp.int32)
    fallback = jnp.full((BB, 1), 0, dtype=jnp.int32)
    best_ll = jnp.full((BB, 1), ninf, dtype=jnp.float32)
    for e in range(E):
        x = logits_refs[e][...]
        m = jnp.max(x, axis=1, keepdims=True)
        ll = jnp.max(jnp.where(lab_eq, x, ninf), axis=1, keepdims=True)
        pm = jnp.max(jnp.where(pre, x, ninf), axis=1, keepdims=True)
        correct = (ll >= m) & (pm < m)
        first_correct = jnp.where(
            (first_correct == E) & correct, e, first_correct
        )
        take = ll > best_ll
        fallback = jnp.where(take, e, fallback)
        best_ll = jnp.where(take, ll, best_ll)

    best = jnp.where(first_correct < E, first_correct, fallback)

    out = logits_refs[0][...]
    for e in range(1, E):
        out = jnp.where(best == e, logits_refs[e][...], out)
    out_ref[...] = out


@jax.jit
def kernel(labels, logits_0, logits_1, logits_2, logits_3, logits_4,
           logits_5, logits_6, logits_7):
    labels2 = labels.reshape(B, 1).astype(jnp.int32)
    logit_spec = pl.BlockSpec((BB, L), lambda i: (i, 0))
    grid = (B // BB,)
    out = pl.pallas_call(
        _oracle_block,
        grid=grid,
        in_specs=[pl.BlockSpec((BB, 1), lambda i: (i, 0))] + [logit_spec] * E,
        out_specs=logit_spec,
        out_shape=jax.ShapeDtypeStruct((B, L), jnp.float32),
        compiler_params=pltpu.CompilerParams(
            dimension_semantics=("arbitrary",),
        ),
    )(labels2, logits_0, logits_1, logits_2, logits_3, logits_4,
      logits_5, logits_6, logits_7)
    return out
